# Initial kernel scaffold; baseline (speedup 1.0000x reference)
import jax
import jax.numpy as jnp
from jax.experimental import pallas as pl


def kernel(x, expert_indices, w1, w2, w3, scales1, scales2, scales3):
    raise NotImplementedError("write your pallas kernel here")


# speedup vs baseline: 5.5359x; 5.5359x over previous
import functools

import jax
import jax.numpy as jnp
from jax.experimental import pallas as pl
from jax.experimental.pallas import tpu as pltpu

E, I, D, T, A = 8, 4096, 1024, 8, 2
IB = 512
K = I // IB


def _ffn_kernel(idx_ref,
                x_ref,
                w1_ref, w3_ref,
                w2_ref,
                s1_ref, s3_ref,
                s2_ref,
                out_ref,
                acc_ref):
    e = pl.program_id(0)
    k = pl.program_id(1)

    xb = x_ref[...].astype(jnp.bfloat16)
    w1 = w1_ref[0].astype(jnp.bfloat16)
    w3 = w3_ref[0].astype(jnp.bfloat16)
    dimn = (((1,), (1,)), ((), ()))
    h1 = jax.lax.dot_general(xb, w1, dimn,
                             preferred_element_type=jnp.float32)
    h3 = jax.lax.dot_general(xb, w3, dimn,
                             preferred_element_type=jnp.float32)
    s1 = s1_ref[0]
    s3 = s3_ref[0]
    g1 = h1 * s1
    x1 = g1 * jax.lax.logistic(g1)
    g = (x1 * (h3 * s3)).astype(jnp.bfloat16)

    w2 = w2_ref[0].astype(jnp.bfloat16)
    y = jax.lax.dot_general(g, w2, dimn,
                            preferred_element_type=jnp.float32)

    @pl.when(k == 0)
    def _():
        acc_ref[...] = y

    @pl.when(k > 0)
    def _():
        acc_ref[...] += y

    @pl.when(k == K - 1)
    def _():
        yo = acc_ref[...] * s2_ref[0]
        for t in range(T):
            for a in range(A):
                @pl.when(idx_ref[t, a] == e)
                def _():
                    out_ref[t, a, :] = yo[t, :]


@jax.jit
def kernel(x, expert_indices, w1, w2, w3, scales1, scales2, scales3):
    idx = expert_indices.astype(jnp.int32)
    s1r = scales1.reshape(E * K, 1, IB)
    s3r = scales3.reshape(E * K, 1, IB)
    s2r = scales2.reshape(E, 1, D)
    grid = (E, K)
    out = pl.pallas_call(
        _ffn_kernel,
        grid=grid,
        in_specs=[
            pl.BlockSpec(memory_space=pltpu.SMEM),
            pl.BlockSpec((T, D), lambda e, k: (0, 0)),
            pl.BlockSpec((1, IB, D), lambda e, k: (e, k, 0)),
            pl.BlockSpec((1, IB, D), lambda e, k: (e, k, 0)),
            pl.BlockSpec((1, D, IB), lambda e, k: (e, 0, k)),
            pl.BlockSpec((1, 1, IB), lambda e, k: (e * K + k, 0, 0)),
            pl.BlockSpec((1, 1, IB), lambda e, k: (e * K + k, 0, 0)),
            pl.BlockSpec((1, 1, D), lambda e, k: (e, 0, 0)),
        ],
        out_specs=pl.BlockSpec((T, A, D), lambda e, k: (0, 0, 0)),
        out_shape=jax.ShapeDtypeStruct((T, A, D), jnp.float32),
        scratch_shapes=[pltpu.VMEM((T, D), jnp.float32)],
    )(idx, x, w1, w3, w2, s1r, s3r, s2r)
    return out


# speedup vs baseline: 6.9286x; 1.2516x over previous
import functools

import jax
import jax.numpy as jnp
from jax.experimental import pallas as pl
from jax.experimental.pallas import tpu as pltpu

E, I, D, T, A = 8, 4096, 1024, 8, 2
IB = 1024
K = I // IB


def _ffn_kernel(idx_ref,
                x_ref,
                w1_ref, w3_ref,
                w2_ref,
                s1_ref, s3_ref,
                s2_ref,
                out_ref,
                acc_ref):
    e = pl.program_id(0)
    k = pl.program_id(1)

    xb = x_ref[...].astype(jnp.bfloat16)
    w1 = w1_ref[0].astype(jnp.bfloat16)
    w3 = w3_ref[0].astype(jnp.bfloat16)
    dimn = (((1,), (1,)), ((), ()))
    h1 = jax.lax.dot_general(xb, w1, dimn,
                             preferred_element_type=jnp.float32)
    h3 = jax.lax.dot_general(xb, w3, dimn,
                             preferred_element_type=jnp.float32)
    s1 = s1_ref[0]
    s3 = s3_ref[0]
    g1 = h1 * s1
    x1 = g1 * jax.lax.logistic(g1)
    g = (x1 * (h3 * s3)).astype(jnp.bfloat16)

    w2 = w2_ref[0].astype(jnp.bfloat16)
    y = jax.lax.dot_general(g, w2, dimn,
                            preferred_element_type=jnp.float32)

    @pl.when(k == 0)
    def _():
        acc_ref[...] = y

    @pl.when(k > 0)
    def _():
        acc_ref[...] += y

    @pl.when(k == K - 1)
    def _():
        yo = acc_ref[...] * s2_ref[0]
        for t in range(T):
            for a in range(A):
                @pl.when(idx_ref[t, a] == e)
                def _():
                    out_ref[t, a, :] = yo[t, :]


@jax.jit
def kernel(x, expert_indices, w1, w2, w3, scales1, scales2, scales3):
    idx = expert_indices.astype(jnp.int32)
    s1r = scales1.reshape(E * K, 1, IB)
    s3r = scales3.reshape(E * K, 1, IB)
    s2r = scales2.reshape(E, 1, D)
    grid = (E, K)
    out = pl.pallas_call(
        _ffn_kernel,
        grid=grid,
        in_specs=[
            pl.BlockSpec(memory_space=pltpu.SMEM),
            pl.BlockSpec((T, D), lambda e, k: (0, 0)),
            pl.BlockSpec((1, IB, D), lambda e, k: (e, k, 0)),
            pl.BlockSpec((1, IB, D), lambda e, k: (e, k, 0)),
            pl.BlockSpec((1, D, IB), lambda e, k: (e, 0, k)),
            pl.BlockSpec((1, 1, IB), lambda e, k: (e * K + k, 0, 0)),
            pl.BlockSpec((1, 1, IB), lambda e, k: (e * K + k, 0, 0)),
            pl.BlockSpec((1, 1, D), lambda e, k: (e, 0, 0)),
        ],
        out_specs=pl.BlockSpec((T, A, D), lambda e, k: (0, 0, 0)),
        out_shape=jax.ShapeDtypeStruct((T, A, D), jnp.float32),
        scratch_shapes=[pltpu.VMEM((T, D), jnp.float32)],
    )(idx, x, w1, w3, w2, s1r, s3r, s2r)
    return out


# speedup vs baseline: 7.2103x; 1.0406x over previous
import functools

import jax
import jax.numpy as jnp
from jax.experimental import pallas as pl
from jax.experimental.pallas import tpu as pltpu

E, I, D, T, A = 8, 4096, 1024, 8, 2
IB = 2048
K = I // IB


def _ffn_kernel(idx_ref,
                x_ref,
                w1_ref, w3_ref,
                w2_ref,
                s1_ref, s3_ref,
                s2_ref,
                out_ref,
                acc_ref):
    e = pl.program_id(0)
    k = pl.program_id(1)

    xb = x_ref[...].astype(jnp.bfloat16)
    w1 = w1_ref[0].astype(jnp.bfloat16)
    w3 = w3_ref[0].astype(jnp.bfloat16)
    dimn = (((1,), (1,)), ((), ()))
    h1 = jax.lax.dot_general(xb, w1, dimn,
                             preferred_element_type=jnp.float32)
    h3 = jax.lax.dot_general(xb, w3, dimn,
                             preferred_element_type=jnp.float32)
    s1 = s1_ref[0]
    s3 = s3_ref[0]
    g1 = h1 * s1
    x1 = g1 * jax.lax.logistic(g1)
    g = (x1 * (h3 * s3)).astype(jnp.bfloat16)

    w2 = w2_ref[0].astype(jnp.bfloat16)
    y = jax.lax.dot_general(g, w2, dimn,
                            preferred_element_type=jnp.float32)

    @pl.when(k == 0)
    def _():
        acc_ref[...] = y

    @pl.when(k > 0)
    def _():
        acc_ref[...] += y

    @pl.when(k == K - 1)
    def _():
        yo = acc_ref[...] * s2_ref[0]
        for t in range(T):
            for a in range(A):
                @pl.when(idx_ref[t, a] == e)
                def _():
                    out_ref[t, a, :] = yo[t, :]


@jax.jit
def kernel(x, expert_indices, w1, w2, w3, scales1, scales2, scales3):
    idx = expert_indices.astype(jnp.int32)
    s1r = scales1.reshape(E * K, 1, IB)
    s3r = scales3.reshape(E * K, 1, IB)
    s2r = scales2.reshape(E, 1, D)
    grid = (E, K)
    out = pl.pallas_call(
        _ffn_kernel,
        grid=grid,
        in_specs=[
            pl.BlockSpec(memory_space=pltpu.SMEM),
            pl.BlockSpec((T, D), lambda e, k: (0, 0)),
            pl.BlockSpec((1, IB, D), lambda e, k: (e, k, 0)),
            pl.BlockSpec((1, IB, D), lambda e, k: (e, k, 0)),
            pl.BlockSpec((1, D, IB), lambda e, k: (e, 0, k)),
            pl.BlockSpec((1, 1, IB), lambda e, k: (e * K + k, 0, 0)),
            pl.BlockSpec((1, 1, IB), lambda e, k: (e * K + k, 0, 0)),
            pl.BlockSpec((1, 1, D), lambda e, k: (e, 0, 0)),
        ],
        out_specs=pl.BlockSpec((T, A, D), lambda e, k: (0, 0, 0)),
        out_shape=jax.ShapeDtypeStruct((T, A, D), jnp.float32),
        scratch_shapes=[pltpu.VMEM((T, D), jnp.float32)],
    )(idx, x, w1, w3, w2, s1r, s3r, s2r)
    return out
